# Initial kernel scaffold; baseline (speedup 1.0000x reference)
import jax
import jax.numpy as jnp
from jax.experimental import pallas as pl


def kernel(center, outside, negative, W_center, W_outside):
    raise NotImplementedError("write your pallas kernel here")


# speedup vs baseline: 2.9670x; 2.9670x over previous
import jax
import jax.numpy as jnp
from jax import lax
from jax.experimental import pallas as pl
from jax.experimental.pallas import tpu as pltpu
from jax.experimental.pallas import tpu_sc as plsc

V = 1_000_000
D = 64
B = 16384
K = 10
L = 16
NC = 2
NS = 16
NW = NC * NS
BPW = B // NW
CHUNK = 128
NCH = BPW // CHUNK
DV = D // L


def _sc_body(wc_hbm, wo_hbm, cidx_hbm, oidx_hbm, nidx_hbm,
             po_hbm, pn_hbm,
             cidx_v, oidx_v, nidx_v, crows_v, orows_v, nsum_v, po_v, pn_v,
             csem, osem, nsem):
    wid = lax.axis_index("s") * NC + lax.axis_index("c")
    row0 = wid * NCH

    pltpu.sync_copy(cidx_hbm.at[pl.ds(row0, NCH)], cidx_v)
    pltpu.sync_copy(oidx_hbm.at[pl.ds(row0, NCH)], oidx_v)
    pltpu.sync_copy(nidx_hbm.at[:, pl.ds(row0, NCH), :], nidx_v)

    co_descs = []
    for j in range(NCH):
        co_descs.append(pltpu.async_copy(
            wc_hbm.at[cidx_v.at[j]], crows_v.at[pl.ds(j * CHUNK, CHUNK)], csem))
        co_descs.append(pltpu.async_copy(
            wo_hbm.at[oidx_v.at[j]], orows_v.at[pl.ds(j * CHUNK, CHUNK)], osem))

    for k in range(K):
        descs = []
        for j in range(NCH):
            descs.append(pltpu.async_copy(
                wo_hbm.at[nidx_v.at[k, j]], nsum_v.at[pl.ds(j * CHUNK, CHUNK)],
                nsem, add=(k > 0)))
        for d in descs:
            d.wait()

    for d in co_descs:
        d.wait()

    def body(r, carry):
        acc_o = None
        acc_n = None
        for j in range(DV):
            c = crows_v[r, pl.ds(j * L, L)]
            o = orows_v[r, pl.ds(j * L, L)]
            n = nsum_v[r, pl.ds(j * L, L)]
            po = c * o
            pn = c * n
            acc_o = po if acc_o is None else acc_o + po
            acc_n = pn if acc_n is None else acc_n + pn
        po_v[r, :] = acc_o
        pn_v[r, :] = acc_n
        return carry

    lax.fori_loop(0, BPW, body, 0, unroll=2)

    b0 = wid * BPW
    pltpu.sync_copy(po_v, po_hbm.at[pl.ds(b0, BPW)])
    pltpu.sync_copy(pn_v, pn_hbm.at[pl.ds(b0, BPW)])


_sc_call = pl.kernel(
    _sc_body,
    out_type=(jax.ShapeDtypeStruct((B, L), jnp.float32),
              jax.ShapeDtypeStruct((B, L), jnp.float32)),
    mesh=plsc.VectorSubcoreMesh(core_axis_name="c", subcore_axis_name="s"),
    compiler_params=pltpu.CompilerParams(use_tc_tiling_on_sc=False),
    scratch_types=[
        pltpu.VMEM((NCH, CHUNK), jnp.int32),
        pltpu.VMEM((NCH, CHUNK), jnp.int32),
        pltpu.VMEM((K, NCH, CHUNK), jnp.int32),
        pltpu.VMEM((BPW, D), jnp.float32),
        pltpu.VMEM((BPW, D), jnp.float32),
        pltpu.VMEM((BPW, D), jnp.float32),
        pltpu.VMEM((BPW, L), jnp.float32),
        pltpu.VMEM((BPW, L), jnp.float32),
        pltpu.SemaphoreType.DMA,
        pltpu.SemaphoreType.DMA,
        pltpu.SemaphoreType.DMA,
    ],
)


def _tc_body(po_ref, pn_ref, out_ref):
    po = po_ref[...]
    pn = pn_ref[...]
    u = jnp.sum(po, axis=1, keepdims=True)
    s = jnp.sum(pn, axis=1, keepdims=True)

    def logsig(x):
        return jnp.minimum(x, 0.0) - jnp.log1p(jnp.exp(-jnp.abs(x)))

    loss = logsig(u) + logsig(-s)
    out_ref[0, 0] = -jnp.sum(loss) / B


_tc_call = pl.pallas_call(
    _tc_body,
    out_shape=jax.ShapeDtypeStruct((1, 1), jnp.float32),
    out_specs=pl.BlockSpec(memory_space=pltpu.SMEM),
)


def kernel(center, outside, negative, W_center, W_outside):
    c = center.reshape(B).astype(jnp.int32).reshape(B // CHUNK, CHUNK)
    o = outside.reshape(B).astype(jnp.int32).reshape(B // CHUNK, CHUNK)
    n = negative.astype(jnp.int32).T.reshape(K, B // CHUNK, CHUNK)
    po, pn = _sc_call(W_center, W_outside, c, o, n)
    return _tc_call(po, pn)[0, 0]


# speedup vs baseline: 3.1158x; 1.0501x over previous
import jax
import jax.numpy as jnp
from jax import lax
from jax.experimental import pallas as pl
from jax.experimental.pallas import tpu as pltpu
from jax.experimental.pallas import tpu_sc as plsc

V = 1_000_000
D = 64
B = 16384
K = 10
L = 16
W128 = 128
NC = 2
NS = 16
NW = NC * NS
BPW = B // NW
CHUNK = 128
NCH = BPW // CHUNK
HALF = BPW // 2
HCH = NCH // 2
DV = D // L


def _sc_body(wc_hbm, wo_hbm, cidx_hbm, oidx_hbm, nidx_hbm,
             out_hbm,
             cidx_v, oidx_v, nidx_v, crows_v, orows_v, nsum_v,
             csem, osem, nsem):
    wid = lax.axis_index("s") * NC + lax.axis_index("c")
    row0 = wid * NCH

    pltpu.sync_copy(cidx_hbm.at[pl.ds(row0, NCH)], cidx_v)
    pltpu.sync_copy(oidx_hbm.at[pl.ds(row0, NCH)], oidx_v)
    pltpu.sync_copy(nidx_hbm.at[:, pl.ds(row0, NCH), :], nidx_v)

    for h in range(2):
        co_descs = []
        for jj in range(HCH):
            j = h * HCH + jj
            co_descs.append(pltpu.async_copy(
                wc_hbm.at[cidx_v.at[j]],
                crows_v.at[pl.ds(jj * CHUNK, CHUNK)], csem))
            co_descs.append(pltpu.async_copy(
                wo_hbm.at[oidx_v.at[j]],
                orows_v.at[pl.ds(jj * CHUNK, CHUNK)], osem))

        for k in range(K):
            descs = []
            for jj in range(HCH):
                j = h * HCH + jj
                descs.append(pltpu.async_copy(
                    wo_hbm.at[nidx_v.at[k, j]],
                    nsum_v.at[pl.ds(jj * CHUNK, CHUNK)],
                    nsem, add=(k > 0)))
            for d in descs:
                d.wait()

        for d in co_descs:
            d.wait()

        def body(r, carry):
            acc_o = None
            acc_n = None
            for j in range(DV):
                c = crows_v[r, pl.ds(j * L, L)]
                o = orows_v[r, pl.ds(j * L, L)]
                n = nsum_v[r, pl.ds(j * L, L)]
                po = c * o
                pn = c * n
                acc_o = po if acc_o is None else acc_o + po
                acc_n = pn if acc_n is None else acc_n + pn
            crows_v[r, pl.ds(0, L)] = acc_o
            crows_v[r, pl.ds(L, L)] = acc_n
            return carry

        lax.fori_loop(0, HALF, body, 0, unroll=2)

        b0 = wid * BPW + h * HALF
        pltpu.sync_copy(crows_v, out_hbm.at[pl.ds(b0, HALF)])


_sc_call = pl.kernel(
    _sc_body,
    out_type=jax.ShapeDtypeStruct((B, W128), jnp.float32),
    mesh=plsc.VectorSubcoreMesh(core_axis_name="c", subcore_axis_name="s"),
    compiler_params=pltpu.CompilerParams(use_tc_tiling_on_sc=True),
    scratch_types=[
        pltpu.VMEM((NCH, CHUNK), jnp.int32),
        pltpu.VMEM((NCH, CHUNK), jnp.int32),
        pltpu.VMEM((K, NCH, CHUNK), jnp.int32),
        pltpu.VMEM((HALF, W128), jnp.float32),
        pltpu.VMEM((HALF, W128), jnp.float32),
        pltpu.VMEM((HALF, W128), jnp.float32),
        pltpu.SemaphoreType.DMA,
        pltpu.SemaphoreType.DMA,
        pltpu.SemaphoreType.DMA,
    ],
)


def _tc_body(x_ref, out_ref):
    x = x_ref[...]
    u = jnp.sum(x[:, 0:L], axis=1, keepdims=True)
    s = jnp.sum(x[:, L:2 * L], axis=1, keepdims=True)

    def logsig(t):
        return jnp.minimum(t, 0.0) - jnp.log1p(jnp.exp(-jnp.abs(t)))

    loss = logsig(u) + logsig(-s)
    out_ref[0, 0] = -jnp.sum(loss) / B


_tc_call = pl.pallas_call(
    _tc_body,
    out_shape=jax.ShapeDtypeStruct((1, 1), jnp.float32),
    out_specs=pl.BlockSpec(memory_space=pltpu.SMEM),
)


def kernel(center, outside, negative, W_center, W_outside):
    wc = jnp.pad(W_center, ((0, 0), (0, W128 - D)))
    wo = jnp.pad(W_outside, ((0, 0), (0, W128 - D)))
    c = center.reshape(B).astype(jnp.int32).reshape(B // CHUNK, CHUNK)
    o = outside.reshape(B).astype(jnp.int32).reshape(B // CHUNK, CHUNK)
    n = negative.astype(jnp.int32).T.reshape(K, B // CHUNK, CHUNK)
    parts = _sc_call(wc, wo, c, o, n)
    return _tc_call(parts)[0, 0]
